# baseline (device time: 14931 ns/iter reference)
import jax
import jax.numpy as jnp
from jax import lax
from jax.experimental import pallas as pl
from jax.experimental.pallas import tpu as pltpu

Q = 256
H = 128


def kernel(x):
    m_per, n = x.shape

    def body(x_ref, out_ref, xv, mine_v, recv_v, send_sems, recv_sems,
             copy_sems):
        my_x = lax.axis_index("x")
        my_y = lax.axis_index("y")
        my_z = lax.axis_index("z")
        k = 2 * my_y + my_z
        ka = 3 - k
        base_mine = my_x * m_per
        base_rem = (1 - my_x) * m_per
        xp = (1 - my_x, my_y, my_z)
        yp = (my_x, 1 - my_y, my_z)
        zp = (my_x, my_y, 1 - my_z)

        barrier = pltpu.get_barrier_semaphore()
        for dev in (xp, yp, zp):
            pl.semaphore_signal(barrier, inc=1, device_id=dev,
                                device_id_type=pltpu.DeviceIdType.MESH)

        stage = pltpu.make_async_copy(x_ref, xv, copy_sems.at[0])
        stage.start()
        stage.wait()
        mine_v[pl.ds(k * Q, Q), :] = xv[pl.ds(k * Q, Q), :].astype(
            jnp.bfloat16)

        pl.semaphore_wait(barrier, 3)

        def rcopy(src_rows, dst_rows, nrows, sem_i, dev):
            return pltpu.make_async_remote_copy(
                src_ref=mine_v.at[pl.ds(src_rows, nrows), :]
                if dev is xp else recv_v.at[pl.ds(src_rows, nrows), :],
                dst_ref=recv_v.at[pl.ds(dst_rows, nrows), :],
                send_sem=send_sems.at[sem_i],
                recv_sem=recv_sems.at[sem_i],
                device_id=dev,
                device_id_type=pltpu.DeviceIdType.MESH,
            )

        a1 = rcopy(k * Q, k * Q, H, 0, xp)
        a2 = rcopy(k * Q + H, k * Q + H, H, 1, xp)
        a1.start()
        a2.start()
        mine_v[pl.ds(ka * Q, Q), :] = xv[pl.ds(ka * Q, Q), :].astype(
            jnp.bfloat16)
        a3 = rcopy(ka * Q, ka * Q, Q, 2, xp)
        a3.start()

        kb = 2 * (1 - my_y) + my_z
        kc = 2 * my_y + (1 - my_z)
        mine_v[pl.ds(kb * Q, Q), :] = xv[pl.ds(kb * Q, Q), :].astype(
            jnp.bfloat16)
        mine_v[pl.ds(kc * Q, Q), :] = xv[pl.ds(kc * Q, Q), :].astype(
            jnp.bfloat16)
        mine_out = pltpu.make_async_copy(
            mine_v, out_ref.at[pl.ds(base_mine, m_per), :], copy_sems.at[1])
        mine_out.start()

        def drain(rows, nrows, sem_i):
            cp = pltpu.make_async_copy(
                recv_v.at[pl.ds(rows, nrows), :],
                out_ref.at[pl.ds(base_rem + rows, nrows), :],
                copy_sems.at[sem_i],
            )
            cp.start()
            return cp

        a1.wait_recv()
        fy1 = rcopy(k * Q, k * Q, H, 3, yp)
        fz1 = rcopy(k * Q, k * Q, H, 4, zp)
        fy1.start()
        fz1.start()
        d1 = drain(k * Q, H, 2)
        a2.wait_recv()
        fy2 = rcopy(k * Q + H, k * Q + H, H, 5, yp)
        fz2 = rcopy(k * Q + H, k * Q + H, H, 6, zp)
        fy2.start()
        fz2.start()
        d2 = drain(k * Q + H, H, 3)

        fy1.wait_recv()
        d3 = drain(kb * Q, H, 4)
        fz1.wait_recv()
        d4 = drain(kc * Q, H, 5)
        fy2.wait_recv()
        d5 = drain(kb * Q + H, H, 6)
        fz2.wait_recv()
        d6 = drain(kc * Q + H, H, 7)
        a3.wait_recv()
        d7 = drain(ka * Q, Q, 8)

        mine_out.wait()
        for d in (d1, d2, d3, d4, d5, d6, d7):
            d.wait()
        a1.wait_send()
        a2.wait_send()
        a3.wait_send()
        fy1.wait_send()
        fz1.wait_send()
        fy2.wait_send()
        fz2.wait_send()

    return pl.pallas_call(
        body,
        out_shape=jax.ShapeDtypeStruct((2 * m_per, n), jnp.bfloat16),
        in_specs=[pl.BlockSpec(memory_space=pl.ANY)],
        out_specs=pl.BlockSpec(memory_space=pl.ANY),
        scratch_shapes=[
            pltpu.VMEM((m_per, n), x.dtype),
            pltpu.VMEM((m_per, n), jnp.bfloat16),
            pltpu.VMEM((m_per, n), jnp.bfloat16),
            pltpu.SemaphoreType.DMA((7,)),
            pltpu.SemaphoreType.DMA((7,)),
            pltpu.SemaphoreType.DMA((9,)),
        ],
        compiler_params=pltpu.CompilerParams(collective_id=0),
    )(x)
